# Initial kernel scaffold; baseline (speedup 1.0000x reference)
import jax
import jax.numpy as jnp
from jax import lax
from jax.experimental import pallas as pl
from jax.experimental.pallas import tpu as pltpu

B = 32
C = 64
P = 512
K = 1024
N_TOK = B * P


def _vq_body(z_ref, cb_ref, zq_ref, idx_ref, loss_ref, ppl_ref,
             loss_acc, cnt_acc):
    b = pl.program_id(0)
    nb = pl.num_programs(0)

    @pl.when(b == 0)
    def _init():
        loss_acc[0, 0] = 0.0
        cnt_acc[...] = jnp.zeros_like(cnt_acc)

    zb = z_ref[0]
    cb = cb_ref[...]

    scoresT = lax.dot_general(cb, zb, (((1,), (0,)), ((), ())),
                              preferred_element_type=jnp.float32)
    rn = jnp.sum(zb * zb, axis=0, keepdims=True)
    cn = jnp.sum(cb * cb, axis=1, keepdims=True)
    d2 = (rn - 2.0 * scoresT) + cn

    m = jnp.min(d2, axis=0, keepdims=True)
    kio = lax.broadcasted_iota(jnp.int32, (K, P), 0)
    idx = jnp.min(jnp.where(d2 == m, kio, K), axis=0)

    onehot = (kio == idx[None, :]).astype(jnp.float32)
    zqT = lax.dot_general(cb, onehot, (((0,), (0,)), ((), ())),
                          preferred_element_type=jnp.float32,
                          precision=lax.Precision.HIGHEST)
    zq_ref[0] = zb + (zqT - zb)
    idx_ref[0, 0] = idx

    loss_acc[0, 0] += jnp.sum(m)
    cnt_acc[...] += jnp.sum(onehot, axis=1, keepdims=True)

    @pl.when(b == nb - 1)
    def _fini():
        loss_ref[0, 0] = 0.25 * loss_acc[0, 0] / (N_TOK * C)
        p = cnt_acc[...] * (1.0 / N_TOK)
        ent = jnp.sum(p * jnp.log(p + 1e-10))
        ppl_ref[0, 0] = jnp.exp(-ent)


def kernel(z, codebook):
    zr = z.reshape(B, C, P)
    zq, idx, loss, ppl = pl.pallas_call(
        _vq_body,
        grid=(B,),
        in_specs=[
            pl.BlockSpec((1, C, P), lambda b: (b, 0, 0)),
            pl.BlockSpec((K, C), lambda b: (0, 0)),
        ],
        out_specs=[
            pl.BlockSpec((1, C, P), lambda b: (b, 0, 0)),
            pl.BlockSpec((1, 1, P), lambda b: (b, 0, 0)),
            pl.BlockSpec((1, 1), lambda b: (0, 0)),
            pl.BlockSpec((1, 1), lambda b: (0, 0)),
        ],
        out_shape=[
            jax.ShapeDtypeStruct((B, C, P), jnp.float32),
            jax.ShapeDtypeStruct((B, 1, P), jnp.int32),
            jax.ShapeDtypeStruct((1, 1), jnp.float32),
            jax.ShapeDtypeStruct((1, 1), jnp.float32),
        ],
        scratch_shapes=[
            pltpu.VMEM((1, 1), jnp.float32),
            pltpu.VMEM((K, 1), jnp.float32),
        ],
    )(zr, codebook)
    z_q = zq.reshape(B, C, 8, 8, 8)
    indices = idx.reshape(B, 8, 8, 8)
    return z_q, loss.reshape(()), indices, ppl.reshape(())


# speedup vs baseline: 1.2120x; 1.2120x over previous
import jax
import jax.numpy as jnp
from jax import lax
from jax.experimental import pallas as pl
from jax.experimental.pallas import tpu as pltpu

B = 32
C = 64
P = 512
K = 1024
N_TOK = B * P


def _vq_body(z_ref, cb_ref, zq_ref, idx_ref, loss_ref, ppl_ref,
             loss_acc, cnt_acc):
    b = pl.program_id(0)
    nb = pl.num_programs(0)

    @pl.when(b == 0)
    def _init():
        loss_acc[...] = jnp.zeros_like(loss_acc)
        cnt_acc[...] = jnp.zeros_like(cnt_acc)

    zb = z_ref[0]
    cb = cb_ref[...]

    scoresT = lax.dot_general(cb, zb, (((1,), (0,)), ((), ())),
                              preferred_element_type=jnp.float32)
    rn = jnp.sum(zb * zb, axis=0, keepdims=True)
    cn = jnp.sum(cb * cb, axis=1, keepdims=True)
    d2 = (rn - 2.0 * scoresT) + cn

    m = jnp.min(d2, axis=0, keepdims=True)
    kio = lax.broadcasted_iota(jnp.int32, (K, P), 0)
    idx = jnp.min(jnp.where(d2 == m, kio, K), axis=0)

    onehot = (kio == idx[None, :]).astype(jnp.float32)
    zqT = lax.dot_general(cb, onehot, (((0,), (0,)), ((), ())),
                          preferred_element_type=jnp.float32,
                          precision=lax.Precision.HIGHEST)
    zq_ref[0] = zb + (zqT - zb)
    idx_ref[0, 0] = idx

    loss_acc[...] += jnp.sum(m, keepdims=True)
    cnt_acc[...] += jnp.sum(onehot, axis=1, keepdims=True)

    @pl.when(b == nb - 1)
    def _fini():
        loss_ref[...] = loss_acc[...] * (0.25 / (N_TOK * C))
        p = cnt_acc[...] * (1.0 / N_TOK)
        ent = jnp.sum(p * jnp.log(p + 1e-10), keepdims=True)
        ppl_ref[...] = jnp.exp(-ent)


def kernel(z, codebook):
    zr = z.reshape(B, C, P)
    zq, idx, loss, ppl = pl.pallas_call(
        _vq_body,
        grid=(B,),
        in_specs=[
            pl.BlockSpec((1, C, P), lambda b: (b, 0, 0)),
            pl.BlockSpec((K, C), lambda b: (0, 0)),
        ],
        out_specs=[
            pl.BlockSpec((1, C, P), lambda b: (b, 0, 0)),
            pl.BlockSpec((1, 1, P), lambda b: (b, 0, 0)),
            pl.BlockSpec((1, 1), lambda b: (0, 0)),
            pl.BlockSpec((1, 1), lambda b: (0, 0)),
        ],
        out_shape=[
            jax.ShapeDtypeStruct((B, C, P), jnp.float32),
            jax.ShapeDtypeStruct((B, 1, P), jnp.int32),
            jax.ShapeDtypeStruct((1, 1), jnp.float32),
            jax.ShapeDtypeStruct((1, 1), jnp.float32),
        ],
        scratch_shapes=[
            pltpu.VMEM((1, 1), jnp.float32),
            pltpu.VMEM((K, 1), jnp.float32),
        ],
    )(zr, codebook)
    z_q = zq.reshape(B, C, 8, 8, 8)
    indices = idx.reshape(B, 8, 8, 8)
    return z_q, loss.reshape(()), indices, ppl.reshape(())
